# baseline (speedup 1.0000x reference)
import functools

import jax
import jax.numpy as jnp
from jax.experimental import pallas as pl
from jax.experimental.pallas import tpu as pltpu

_H = 256
_W = 256
_TOPK = 100
_KPAD = 128
_SCALE = 4.0
_THRESH = 0.01
_BIG = 1 << 30


def _decode_body(hm_ref, off_ref, wh_ref, ids_ref, sc_ref, bb_ref, mask_ref):
    h = hm_ref[0]
    ninf = jnp.float32(-jnp.inf)
    col_pad = jnp.full((_H, 1), ninf, jnp.float32)
    row_pad = jnp.full((1, _W), ninf, jnp.float32)
    left = jnp.concatenate([col_pad, h[:, :-1]], axis=1)
    right = jnp.concatenate([h[:, 1:], col_pad], axis=1)
    cm = jnp.maximum(jnp.maximum(left, right), h)
    up = jnp.concatenate([row_pad, cm[:-1, :]], axis=0)
    down = jnp.concatenate([cm[1:, :], row_pad], axis=0)
    pm = jnp.maximum(jnp.maximum(up, down), cm)
    masked = jnp.where(pm == h, h, jnp.float32(0.0))
    mask_ref[...] = masked

    m_t = masked.T
    rowmax = jnp.max(m_t, axis=0, keepdims=True)
    sub_j = jax.lax.broadcasted_iota(jnp.int32, (_W, _H), 0)
    rowarg = jnp.min(jnp.where(m_t == rowmax, sub_j, _BIG), axis=0, keepdims=True)

    lane_i = jax.lax.broadcasted_iota(jnp.int32, (1, _H), 1)
    lane_k = jax.lax.broadcasted_iota(jnp.int32, (1, _KPAD), 1)
    zeros_k = jnp.zeros((1, _KPAD), jnp.float32)

    def step(k, carry):
        rowmax, rowarg, sc_v, fl_v, ox_v, oy_v, w_v, h_v = carry
        gm = jnp.max(rowmax)
        cand = jnp.where(rowmax == gm, lane_i * _W + rowarg, _BIG)
        flat = jnp.min(cand)
        i_s = flat // _W
        j_s = flat % _W
        row = mask_ref[pl.ds(i_s, 1), :]
        newrow = jnp.where(lane_i == j_s, jnp.float32(-1.0), row)
        mask_ref[pl.ds(i_s, 1), :] = newrow
        nm = jnp.max(newrow)
        na = jnp.min(jnp.where(newrow == nm, lane_i, _BIG))
        rowmax = jnp.where(lane_i == i_s, nm, rowmax)
        rowarg = jnp.where(lane_i == i_s, na, rowarg)
        sel = lane_i == j_s
        ox = jnp.sum(jnp.where(sel, off_ref[0, 0, pl.ds(i_s, 1), :], 0.0))
        oy = jnp.sum(jnp.where(sel, off_ref[0, 1, pl.ds(i_s, 1), :], 0.0))
        ww = jnp.sum(jnp.where(sel, wh_ref[0, 0, pl.ds(i_s, 1), :], 0.0))
        hh = jnp.sum(jnp.where(sel, wh_ref[0, 1, pl.ds(i_s, 1), :], 0.0))
        ksel = lane_k == k
        sc_v = jnp.where(ksel, gm, sc_v)
        fl_v = jnp.where(ksel, flat, fl_v)
        ox_v = jnp.where(ksel, ox, ox_v)
        oy_v = jnp.where(ksel, oy, oy_v)
        w_v = jnp.where(ksel, ww, w_v)
        h_v = jnp.where(ksel, hh, h_v)
        return rowmax, rowarg, sc_v, fl_v, ox_v, oy_v, w_v, h_v

    init = (rowmax, rowarg, zeros_k, jnp.zeros((1, _KPAD), jnp.int32),
            zeros_k, zeros_k, zeros_k, zeros_k)
    (_, _, sc_v, fl_v, ox_v, oy_v, w_v, h_v) = jax.lax.fori_loop(
        0, _TOPK, step, init)

    ys = (fl_v // _W).astype(jnp.float32)
    xs = (fl_v % _W).astype(jnp.float32)
    keep = sc_v > _THRESH
    neg1 = jnp.float32(-1.0)
    cx = xs + ox_v
    cy = ys + oy_v
    hw = w_v * 0.5
    hh2 = h_v * 0.5
    x1 = jnp.where(keep, cx - hw, neg1) * _SCALE
    y1 = jnp.where(keep, cy - hh2, neg1) * _SCALE
    x2 = jnp.where(keep, cx + hw, neg1) * _SCALE
    y2 = jnp.where(keep, cy + hh2, neg1) * _SCALE
    ids_ref[0] = jnp.where(keep, jnp.float32(0.0), neg1)
    sc_ref[0] = jnp.where(keep, sc_v, neg1)
    bb_ref[0] = jnp.concatenate([x1, y1, x2, y2], axis=0)


@jax.jit
def kernel(heatmap, offset, wh):
    b = heatmap.shape[0]
    hm = heatmap.reshape(b, _H, _W)
    ids_p, sc_p, bb_p = pl.pallas_call(
        _decode_body,
        grid=(b,),
        in_specs=[
            pl.BlockSpec((1, _H, _W), lambda i: (i, 0, 0)),
            pl.BlockSpec((1, 2, _H, _W), lambda i: (i, 0, 0, 0)),
            pl.BlockSpec((1, 2, _H, _W), lambda i: (i, 0, 0, 0)),
        ],
        out_specs=[
            pl.BlockSpec((1, 1, _KPAD), lambda i: (i, 0, 0)),
            pl.BlockSpec((1, 1, _KPAD), lambda i: (i, 0, 0)),
            pl.BlockSpec((1, 4, _KPAD), lambda i: (i, 0, 0)),
        ],
        out_shape=[
            jax.ShapeDtypeStruct((b, 1, _KPAD), jnp.float32),
            jax.ShapeDtypeStruct((b, 1, _KPAD), jnp.float32),
            jax.ShapeDtypeStruct((b, 4, _KPAD), jnp.float32),
        ],
        scratch_shapes=[pltpu.VMEM((_H, _W), jnp.float32)],
    )(hm, offset, wh)
    ids = ids_p[:, 0, :_TOPK, None]
    scores = sc_p[:, 0, :_TOPK, None]
    bboxes = jnp.transpose(bb_p, (0, 2, 1))[:, :_TOPK, :]
    return ids, scores, bboxes


# speedup vs baseline: 10.3296x; 10.3296x over previous
import functools

import jax
import jax.numpy as jnp
from jax import lax
from jax.experimental import pallas as pl
from jax.experimental.pallas import tpu as pltpu
from jax.experimental.pallas import tpu_sc as plsc

_B = 16
_H = 256
_W = 256
_TOPK = 100
_KPAD = 128
_SCALE = 4.0
_THRESH = 0.01
_BIG = 1 << 30


def _topk_body(hm_ref, sc_ref, fl_ref, mask_ref, rows_ref):
    h = hm_ref[...]
    ninf = jnp.float32(-jnp.inf)
    col_pad = jnp.full((_B, _H, 1), ninf, jnp.float32)
    row_pad = jnp.full((_B, 1, _W), ninf, jnp.float32)
    left = jnp.concatenate([col_pad, h[:, :, :-1]], axis=2)
    right = jnp.concatenate([h[:, :, 1:], col_pad], axis=2)
    cm = jnp.maximum(jnp.maximum(left, right), h)
    up = jnp.concatenate([row_pad, cm[:, :-1, :]], axis=1)
    down = jnp.concatenate([cm[:, 1:, :], row_pad], axis=1)
    pm = jnp.maximum(jnp.maximum(up, down), cm)
    masked = jnp.where(pm == h, h, jnp.float32(0.0))
    mask_ref[...] = masked

    rowmax = jnp.max(masked, axis=2)
    iota_j3 = lax.broadcasted_iota(jnp.int32, (_B, _H, _W), 2)
    rowarg = jnp.min(
        jnp.where(masked == rowmax[:, :, None], iota_j3, _BIG), axis=2)

    lane_i = lax.broadcasted_iota(jnp.int32, (_B, _H), 1)
    batch_i = lax.broadcasted_iota(jnp.int32, (_B, _H), 0)
    lane_k = lax.broadcasted_iota(jnp.int32, (_B, _KPAD), 1)
    lane_1 = lax.broadcasted_iota(jnp.int32, (1, _W), 1)

    def step(k, carry):
        rowmax, rowarg, sc_v, fl_v = carry
        gm = jnp.max(rowmax, axis=1, keepdims=True)
        cand = jnp.where(rowmax == gm, lane_i * _W + rowarg, _BIG)
        flat_v = jnp.min(cand, axis=1, keepdims=True)
        for b in range(_B):
            fb = jnp.min(jnp.where(batch_i == b, cand, _BIG))
            ib = fb // _W
            jb = fb % _W
            row = mask_ref[b, pl.ds(ib, 1), :]
            nr = jnp.where(lane_1 == jb, jnp.float32(-1.0), row)
            mask_ref[b, pl.ds(ib, 1), :] = nr
            rows_ref[pl.ds(b, 1), :] = nr
        newrows = rows_ref[...]
        nm = jnp.max(newrows, axis=1, keepdims=True)
        na = jnp.min(jnp.where(newrows == nm, lane_i, _BIG),
                     axis=1, keepdims=True)
        cond = lane_i == flat_v // _W
        rowmax = jnp.where(cond, nm, rowmax)
        rowarg = jnp.where(cond, na, rowarg)
        ksel = lane_k == k
        sc_v = jnp.where(ksel, gm, sc_v)
        fl_v = jnp.where(ksel, flat_v, fl_v)
        return rowmax, rowarg, sc_v, fl_v

    init = (rowmax, rowarg, jnp.zeros((_B, _KPAD), jnp.float32),
            jnp.zeros((_B, _KPAD), jnp.int32))
    _, _, sc_v, fl_v = lax.fori_loop(0, _TOPK, step, init)
    sc_ref[...] = sc_v
    fl_ref[...] = fl_v


def _topk_call(hm3):
    return pl.pallas_call(
        _topk_body,
        in_specs=[pl.BlockSpec((_B, _H, _W), lambda: (0, 0, 0))],
        out_specs=[
            pl.BlockSpec((_B, _KPAD), lambda: (0, 0)),
            pl.BlockSpec((_B, _KPAD), lambda: (0, 0)),
        ],
        out_shape=[
            jax.ShapeDtypeStruct((_B, _KPAD), jnp.float32),
            jax.ShapeDtypeStruct((_B, _KPAD), jnp.int32),
        ],
        scratch_shapes=[
            pltpu.VMEM((_B, _H, _W), jnp.float32),
            pltpu.VMEM((_B, _W), jnp.float32),
        ],
    )(hm3)


_HALF = 64


def _decode_body(sc_hbm, fl_hbm, off_hbm, wh_hbm, ids_out, sco_out, bb_out,
                 sidx, sscore, pobuf, obuf, wbuf, idsb, scb,
                 x1b, y1b, x2b, y2b, sem1, sem2):
    wid = lax.axis_index("s") * 2 + lax.axis_index("c")
    b = wid // 2
    start = (wid % 2) * _HALF
    pltpu.sync_copy(fl_hbm.at[pl.ds(b * _KPAD + start, _HALF)], sidx)
    pltpu.sync_copy(sc_hbm.at[pl.ds(b * _KPAD + start, _HALF)], sscore)
    base = b * (2 * _H * _W)
    for k in range(_HALF // 16):
        idx = sidx[pl.ds(k * 16, 16)]
        pobuf[pl.ds(k * 16, 16)] = base + idx
        pobuf[pl.ds(_HALF + k * 16, 16)] = base + _H * _W + idx
    pltpu.async_copy(off_hbm.at[pobuf], obuf, sem1).wait()
    pltpu.async_copy(wh_hbm.at[pobuf], wbuf, sem2).wait()
    lane = lax.iota(jnp.int32, 16)
    neg1 = jnp.float32(-1.0)
    for k in range(_HALF // 16):
        idx = sidx[pl.ds(k * 16, 16)]
        s = sscore[pl.ds(k * 16, 16)]
        y = lax.shift_right_logical(idx, 8)
        x = jnp.bitwise_and(idx, _W - 1)
        ox = obuf[pl.ds(k * 16, 16)]
        oy = obuf[pl.ds(_HALF + k * 16, 16)]
        ww = wbuf[pl.ds(k * 16, 16)]
        hh = wbuf[pl.ds(_HALF + k * 16, 16)]
        keep = s > _THRESH
        cx = x.astype(jnp.float32) + ox
        cy = y.astype(jnp.float32) + oy
        hw = ww * 0.5
        hh2 = hh * 0.5
        x1 = jnp.where(keep, cx - hw, neg1) * _SCALE
        y1 = jnp.where(keep, cy - hh2, neg1) * _SCALE
        x2 = jnp.where(keep, cx + hw, neg1) * _SCALE
        y2 = jnp.where(keep, cy + hh2, neg1) * _SCALE
        idsb[pl.ds(k * 16, 16)] = jnp.where(keep, jnp.float32(0.0), neg1)
        scb[pl.ds(k * 16, 16)] = jnp.where(keep, s, neg1)
        x1b[pl.ds(k * 16, 16)] = x1
        y1b[pl.ds(k * 16, 16)] = y1
        x2b[pl.ds(k * 16, 16)] = x2
        y2b[pl.ds(k * 16, 16)] = y2
    pos = b * _KPAD + start
    pltpu.sync_copy(idsb, ids_out.at[pl.ds(pos, _HALF)])
    pltpu.sync_copy(scb, sco_out.at[pl.ds(pos, _HALF)])
    n = _B * _KPAD
    pltpu.sync_copy(x1b, bb_out.at[pl.ds(pos, _HALF)])
    pltpu.sync_copy(y1b, bb_out.at[pl.ds(n + pos, _HALF)])
    pltpu.sync_copy(x2b, bb_out.at[pl.ds(2 * n + pos, _HALF)])
    pltpu.sync_copy(y2b, bb_out.at[pl.ds(3 * n + pos, _HALF)])


def _decode_call(*args):
    return functools.partial(
        pl.kernel,
        mesh=plsc.VectorSubcoreMesh(core_axis_name="c", subcore_axis_name="s"),
        compiler_params=pltpu.CompilerParams(use_tc_tiling_on_sc=False),
        out_type=[
            jax.ShapeDtypeStruct((_B * _KPAD,), jnp.float32),
            jax.ShapeDtypeStruct((_B * _KPAD,), jnp.float32),
            jax.ShapeDtypeStruct((_B * _KPAD * 4,), jnp.float32),
        ],
        scratch_types=[
            pltpu.VMEM((_HALF,), jnp.int32),
            pltpu.VMEM((_HALF,), jnp.float32),
            pltpu.VMEM((2 * _HALF,), jnp.int32),
            pltpu.VMEM((2 * _HALF,), jnp.float32),
            pltpu.VMEM((2 * _HALF,), jnp.float32),
            pltpu.VMEM((_HALF,), jnp.float32),
            pltpu.VMEM((_HALF,), jnp.float32),
            pltpu.VMEM((_HALF,), jnp.float32),
            pltpu.VMEM((_HALF,), jnp.float32),
            pltpu.VMEM((_HALF,), jnp.float32),
            pltpu.VMEM((_HALF,), jnp.float32),
            pltpu.SemaphoreType.DMA,
            pltpu.SemaphoreType.DMA,
        ],
    )(_decode_body)(*args)


@jax.jit
def kernel(heatmap, offset, wh):
    hm3 = heatmap.reshape(_B, _H, _W)
    sc, fl = _topk_call(hm3)
    off_rows = offset.reshape(-1)
    wh_rows = wh.reshape(-1)
    ids_p, sco_p, bb_p = _decode_call(
        sc.reshape(-1), fl.reshape(-1), off_rows, wh_rows)
    ids = ids_p.reshape(_B, _KPAD)[:, :_TOPK, None]
    scores = sco_p.reshape(_B, _KPAD)[:, :_TOPK, None]
    bboxes = jnp.transpose(
        bb_p.reshape(4, _B, _KPAD), (1, 2, 0))[:, :_TOPK, :]
    return ids, scores, bboxes


# speedup vs baseline: 10.7783x; 1.0434x over previous
import functools

import jax
import jax.numpy as jnp
from jax import lax
from jax.experimental import pallas as pl
from jax.experimental.pallas import tpu as pltpu
from jax.experimental.pallas import tpu_sc as plsc

_B = 16
_H = 256
_W = 256
_TOPK = 100
_KPAD = 128
_SCALE = 4.0
_THRESH = 0.01
_BIG = 1 << 30


def _topk_body(hm_ref, sc_ref, fl_ref, rows_ref, *mask_refs):
    h = hm_ref[...]
    ninf = jnp.float32(-jnp.inf)
    col_pad = jnp.full((_B, _H, 1), ninf, jnp.float32)
    row_pad = jnp.full((_B, 1, _W), ninf, jnp.float32)
    left = jnp.concatenate([col_pad, h[:, :, :-1]], axis=2)
    right = jnp.concatenate([h[:, :, 1:], col_pad], axis=2)
    cm = jnp.maximum(jnp.maximum(left, right), h)
    up = jnp.concatenate([row_pad, cm[:, :-1, :]], axis=1)
    down = jnp.concatenate([cm[:, 1:, :], row_pad], axis=1)
    pm = jnp.maximum(jnp.maximum(up, down), cm)
    masked = jnp.where(pm == h, h, jnp.float32(0.0))
    for b in range(_B):
        mask_refs[b][...] = masked[b]

    rowmax = jnp.max(masked, axis=2)
    iota_j3 = lax.broadcasted_iota(jnp.int32, (_B, _H, _W), 2)
    rowarg = jnp.min(
        jnp.where(masked == rowmax[:, :, None], iota_j3, _BIG), axis=2)

    lane_i = lax.broadcasted_iota(jnp.int32, (_B, _H), 1)
    batch_i = lax.broadcasted_iota(jnp.int32, (_B, _H), 0)
    lane_k = lax.broadcasted_iota(jnp.int32, (_B, _KPAD), 1)
    lane_1 = lax.broadcasted_iota(jnp.int32, (1, _W), 1)

    def step(k, carry):
        rowmax, rowarg, sc_v, fl_v = carry
        gm = jnp.max(rowmax, axis=1, keepdims=True)
        cand = jnp.where(rowmax == gm, lane_i * _W + rowarg, _BIG)
        flat_v = jnp.min(cand, axis=1, keepdims=True)
        for b in range(_B):
            fb = jnp.min(jnp.where(batch_i == b, cand, _BIG))
            ib = fb // _W
            jb = fb % _W
            row = mask_refs[b][pl.ds(ib, 1), :]
            nr = jnp.where(lane_1 == jb, jnp.float32(-1.0), row)
            mask_refs[b][pl.ds(ib, 1), :] = nr
            rows_ref[pl.ds(b, 1), :] = nr
        newrows = rows_ref[...]
        nm = jnp.max(newrows, axis=1, keepdims=True)
        na = jnp.min(jnp.where(newrows == nm, lane_i, _BIG),
                     axis=1, keepdims=True)
        cond = lane_i == flat_v // _W
        rowmax = jnp.where(cond, nm, rowmax)
        rowarg = jnp.where(cond, na, rowarg)
        ksel = lane_k == k
        sc_v = jnp.where(ksel, gm, sc_v)
        fl_v = jnp.where(ksel, flat_v, fl_v)
        return rowmax, rowarg, sc_v, fl_v

    init = (rowmax, rowarg, jnp.zeros((_B, _KPAD), jnp.float32),
            jnp.zeros((_B, _KPAD), jnp.int32))
    _, _, sc_v, fl_v = lax.fori_loop(0, _TOPK, step, init)
    sc_ref[...] = sc_v
    fl_ref[...] = fl_v


def _topk_call(hm3):
    return pl.pallas_call(
        _topk_body,
        in_specs=[pl.BlockSpec((_B, _H, _W), lambda: (0, 0, 0))],
        out_specs=[
            pl.BlockSpec((_B, _KPAD), lambda: (0, 0)),
            pl.BlockSpec((_B, _KPAD), lambda: (0, 0)),
        ],
        out_shape=[
            jax.ShapeDtypeStruct((_B, _KPAD), jnp.float32),
            jax.ShapeDtypeStruct((_B, _KPAD), jnp.int32),
        ],
        scratch_shapes=[pltpu.VMEM((_B, _W), jnp.float32)] + [
            pltpu.VMEM((_H, _W), jnp.float32) for _ in range(_B)],
    )(hm3)


_HALF = 64


def _decode_body(sc_hbm, fl_hbm, off_hbm, wh_hbm, ids_out, sco_out, bb_out,
                 sidx, sscore, pobuf, obuf, wbuf, idsb, scb,
                 x1b, y1b, x2b, y2b, sem1, sem2):
    wid = lax.axis_index("s") * 2 + lax.axis_index("c")
    b = wid // 2
    start = (wid % 2) * _HALF
    pltpu.sync_copy(fl_hbm.at[pl.ds(b * _KPAD + start, _HALF)], sidx)
    pltpu.sync_copy(sc_hbm.at[pl.ds(b * _KPAD + start, _HALF)], sscore)
    base = b * (2 * _H * _W)
    for k in range(_HALF // 16):
        idx = sidx[pl.ds(k * 16, 16)]
        pobuf[pl.ds(k * 16, 16)] = base + idx
        pobuf[pl.ds(_HALF + k * 16, 16)] = base + _H * _W + idx
    pltpu.async_copy(off_hbm.at[pobuf], obuf, sem1).wait()
    pltpu.async_copy(wh_hbm.at[pobuf], wbuf, sem2).wait()
    lane = lax.iota(jnp.int32, 16)
    neg1 = jnp.float32(-1.0)
    for k in range(_HALF // 16):
        idx = sidx[pl.ds(k * 16, 16)]
        s = sscore[pl.ds(k * 16, 16)]
        y = lax.shift_right_logical(idx, 8)
        x = jnp.bitwise_and(idx, _W - 1)
        ox = obuf[pl.ds(k * 16, 16)]
        oy = obuf[pl.ds(_HALF + k * 16, 16)]
        ww = wbuf[pl.ds(k * 16, 16)]
        hh = wbuf[pl.ds(_HALF + k * 16, 16)]
        keep = s > _THRESH
        cx = x.astype(jnp.float32) + ox
        cy = y.astype(jnp.float32) + oy
        hw = ww * 0.5
        hh2 = hh * 0.5
        x1 = jnp.where(keep, cx - hw, neg1) * _SCALE
        y1 = jnp.where(keep, cy - hh2, neg1) * _SCALE
        x2 = jnp.where(keep, cx + hw, neg1) * _SCALE
        y2 = jnp.where(keep, cy + hh2, neg1) * _SCALE
        idsb[pl.ds(k * 16, 16)] = jnp.where(keep, jnp.float32(0.0), neg1)
        scb[pl.ds(k * 16, 16)] = jnp.where(keep, s, neg1)
        x1b[pl.ds(k * 16, 16)] = x1
        y1b[pl.ds(k * 16, 16)] = y1
        x2b[pl.ds(k * 16, 16)] = x2
        y2b[pl.ds(k * 16, 16)] = y2
    pos = b * _KPAD + start
    pltpu.sync_copy(idsb, ids_out.at[pl.ds(pos, _HALF)])
    pltpu.sync_copy(scb, sco_out.at[pl.ds(pos, _HALF)])
    n = _B * _KPAD
    pltpu.sync_copy(x1b, bb_out.at[pl.ds(pos, _HALF)])
    pltpu.sync_copy(y1b, bb_out.at[pl.ds(n + pos, _HALF)])
    pltpu.sync_copy(x2b, bb_out.at[pl.ds(2 * n + pos, _HALF)])
    pltpu.sync_copy(y2b, bb_out.at[pl.ds(3 * n + pos, _HALF)])


def _decode_call(*args):
    return functools.partial(
        pl.kernel,
        mesh=plsc.VectorSubcoreMesh(core_axis_name="c", subcore_axis_name="s"),
        compiler_params=pltpu.CompilerParams(use_tc_tiling_on_sc=False),
        out_type=[
            jax.ShapeDtypeStruct((_B * _KPAD,), jnp.float32),
            jax.ShapeDtypeStruct((_B * _KPAD,), jnp.float32),
            jax.ShapeDtypeStruct((_B * _KPAD * 4,), jnp.float32),
        ],
        scratch_types=[
            pltpu.VMEM((_HALF,), jnp.int32),
            pltpu.VMEM((_HALF,), jnp.float32),
            pltpu.VMEM((2 * _HALF,), jnp.int32),
            pltpu.VMEM((2 * _HALF,), jnp.float32),
            pltpu.VMEM((2 * _HALF,), jnp.float32),
            pltpu.VMEM((_HALF,), jnp.float32),
            pltpu.VMEM((_HALF,), jnp.float32),
            pltpu.VMEM((_HALF,), jnp.float32),
            pltpu.VMEM((_HALF,), jnp.float32),
            pltpu.VMEM((_HALF,), jnp.float32),
            pltpu.VMEM((_HALF,), jnp.float32),
            pltpu.SemaphoreType.DMA,
            pltpu.SemaphoreType.DMA,
        ],
    )(_decode_body)(*args)


@jax.jit
def kernel(heatmap, offset, wh):
    hm3 = heatmap.reshape(_B, _H, _W)
    sc, fl = _topk_call(hm3)
    off_rows = offset.reshape(-1)
    wh_rows = wh.reshape(-1)
    ids_p, sco_p, bb_p = _decode_call(
        sc.reshape(-1), fl.reshape(-1), off_rows, wh_rows)
    ids = ids_p.reshape(_B, _KPAD)[:, :_TOPK, None]
    scores = sco_p.reshape(_B, _KPAD)[:, :_TOPK, None]
    bboxes = jnp.transpose(
        bb_p.reshape(4, _B, _KPAD), (1, 2, 0))[:, :_TOPK, :]
    return ids, scores, bboxes


# speedup vs baseline: 10.9469x; 1.0156x over previous
import functools

import jax
import jax.numpy as jnp
from jax import lax
from jax.experimental import pallas as pl
from jax.experimental.pallas import tpu as pltpu
from jax.experimental.pallas import tpu_sc as plsc

_B = 16
_H = 256
_W = 256
_TOPK = 100
_KPAD = 128
_SCALE = 4.0
_THRESH = 0.01
_BIG = 1 << 30


def _topk_body(hm_ref, sc_ref, fl_ref, rows_ref, *mask_refs):
    h = hm_ref[...]
    ninf = jnp.float32(-jnp.inf)
    col_pad = jnp.full((_B, _H, 1), ninf, jnp.float32)
    row_pad = jnp.full((_B, 1, _W), ninf, jnp.float32)
    left = jnp.concatenate([col_pad, h[:, :, :-1]], axis=2)
    right = jnp.concatenate([h[:, :, 1:], col_pad], axis=2)
    cm = jnp.maximum(jnp.maximum(left, right), h)
    up = jnp.concatenate([row_pad, cm[:, :-1, :]], axis=1)
    down = jnp.concatenate([cm[:, 1:, :], row_pad], axis=1)
    pm = jnp.maximum(jnp.maximum(up, down), cm)
    masked = jnp.where(pm == h, h, jnp.float32(0.0))
    for b in range(_B):
        mask_refs[b][...] = masked[b]

    rowmax = jnp.max(masked, axis=2)
    iota_j3 = lax.broadcasted_iota(jnp.int32, (_B, _H, _W), 2)
    rowarg = jnp.min(
        jnp.where(masked == rowmax[:, :, None], iota_j3, _BIG), axis=2)

    lane_i = lax.broadcasted_iota(jnp.int32, (_B, _H), 1)
    batch_16 = lax.broadcasted_iota(jnp.int32, (_B, 1), 0)
    lane_k = lax.broadcasted_iota(jnp.int32, (_B, _KPAD), 1)
    lane_1 = lax.broadcasted_iota(jnp.int32, (1, _W), 1)

    def step(k, carry):
        rowmax, rowarg, sc_v, fl_v = carry
        gm = jnp.max(rowmax, axis=1, keepdims=True)
        cand = jnp.where(rowmax == gm, lane_i * _W + rowarg, _BIG)
        flat_v = jnp.min(cand, axis=1, keepdims=True)
        nrs = []
        for b in range(_B):
            fb = jnp.min(jnp.where(batch_16 == b, flat_v, _BIG))
            ib = fb // _W
            jb = fb % _W
            row = mask_refs[b][pl.ds(ib, 1), :]
            nr = jnp.where(lane_1 == jb, jnp.float32(-1.0), row)
            mask_refs[b][pl.ds(ib, 1), :] = nr
            nrs.append(nr)
        newrows = jnp.concatenate(nrs, axis=0)
        nm = jnp.max(newrows, axis=1, keepdims=True)
        na = jnp.min(jnp.where(newrows == nm, lane_i, _BIG),
                     axis=1, keepdims=True)
        cond = lane_i == flat_v // _W
        rowmax = jnp.where(cond, nm, rowmax)
        rowarg = jnp.where(cond, na, rowarg)
        ksel = lane_k == k
        sc_v = jnp.where(ksel, gm, sc_v)
        fl_v = jnp.where(ksel, flat_v, fl_v)
        return rowmax, rowarg, sc_v, fl_v

    init = (rowmax, rowarg, jnp.zeros((_B, _KPAD), jnp.float32),
            jnp.zeros((_B, _KPAD), jnp.int32))
    _, _, sc_v, fl_v = lax.fori_loop(0, _TOPK, step, init)
    sc_ref[...] = sc_v
    fl_ref[...] = fl_v


def _topk_call(hm3):
    return pl.pallas_call(
        _topk_body,
        in_specs=[pl.BlockSpec((_B, _H, _W), lambda: (0, 0, 0))],
        out_specs=[
            pl.BlockSpec((_B, _KPAD), lambda: (0, 0)),
            pl.BlockSpec((_B, _KPAD), lambda: (0, 0)),
        ],
        out_shape=[
            jax.ShapeDtypeStruct((_B, _KPAD), jnp.float32),
            jax.ShapeDtypeStruct((_B, _KPAD), jnp.int32),
        ],
        scratch_shapes=[pltpu.VMEM((_B, _W), jnp.float32)] + [
            pltpu.VMEM((_H, _W), jnp.float32) for _ in range(_B)],
    )(hm3)


_HALF = 64


def _decode_body(sc_hbm, fl_hbm, off_hbm, wh_hbm, ids_out, sco_out, bb_out,
                 sidx, sscore, pobuf, obuf, wbuf, idsb, scb,
                 x1b, y1b, x2b, y2b, sem1, sem2):
    wid = lax.axis_index("s") * 2 + lax.axis_index("c")
    b = wid // 2
    start = (wid % 2) * _HALF
    pltpu.sync_copy(fl_hbm.at[pl.ds(b * _KPAD + start, _HALF)], sidx)
    pltpu.sync_copy(sc_hbm.at[pl.ds(b * _KPAD + start, _HALF)], sscore)
    base = b * (2 * _H * _W)
    for k in range(_HALF // 16):
        idx = sidx[pl.ds(k * 16, 16)]
        pobuf[pl.ds(k * 16, 16)] = base + idx
        pobuf[pl.ds(_HALF + k * 16, 16)] = base + _H * _W + idx
    pltpu.async_copy(off_hbm.at[pobuf], obuf, sem1).wait()
    pltpu.async_copy(wh_hbm.at[pobuf], wbuf, sem2).wait()
    lane = lax.iota(jnp.int32, 16)
    neg1 = jnp.float32(-1.0)
    for k in range(_HALF // 16):
        idx = sidx[pl.ds(k * 16, 16)]
        s = sscore[pl.ds(k * 16, 16)]
        y = lax.shift_right_logical(idx, 8)
        x = jnp.bitwise_and(idx, _W - 1)
        ox = obuf[pl.ds(k * 16, 16)]
        oy = obuf[pl.ds(_HALF + k * 16, 16)]
        ww = wbuf[pl.ds(k * 16, 16)]
        hh = wbuf[pl.ds(_HALF + k * 16, 16)]
        keep = s > _THRESH
        cx = x.astype(jnp.float32) + ox
        cy = y.astype(jnp.float32) + oy
        hw = ww * 0.5
        hh2 = hh * 0.5
        x1 = jnp.where(keep, cx - hw, neg1) * _SCALE
        y1 = jnp.where(keep, cy - hh2, neg1) * _SCALE
        x2 = jnp.where(keep, cx + hw, neg1) * _SCALE
        y2 = jnp.where(keep, cy + hh2, neg1) * _SCALE
        idsb[pl.ds(k * 16, 16)] = jnp.where(keep, jnp.float32(0.0), neg1)
        scb[pl.ds(k * 16, 16)] = jnp.where(keep, s, neg1)
        x1b[pl.ds(k * 16, 16)] = x1
        y1b[pl.ds(k * 16, 16)] = y1
        x2b[pl.ds(k * 16, 16)] = x2
        y2b[pl.ds(k * 16, 16)] = y2
    pos = b * _KPAD + start
    pltpu.sync_copy(idsb, ids_out.at[pl.ds(pos, _HALF)])
    pltpu.sync_copy(scb, sco_out.at[pl.ds(pos, _HALF)])
    n = _B * _KPAD
    pltpu.sync_copy(x1b, bb_out.at[pl.ds(pos, _HALF)])
    pltpu.sync_copy(y1b, bb_out.at[pl.ds(n + pos, _HALF)])
    pltpu.sync_copy(x2b, bb_out.at[pl.ds(2 * n + pos, _HALF)])
    pltpu.sync_copy(y2b, bb_out.at[pl.ds(3 * n + pos, _HALF)])


def _decode_call(*args):
    return functools.partial(
        pl.kernel,
        mesh=plsc.VectorSubcoreMesh(core_axis_name="c", subcore_axis_name="s"),
        compiler_params=pltpu.CompilerParams(use_tc_tiling_on_sc=False),
        out_type=[
            jax.ShapeDtypeStruct((_B * _KPAD,), jnp.float32),
            jax.ShapeDtypeStruct((_B * _KPAD,), jnp.float32),
            jax.ShapeDtypeStruct((_B * _KPAD * 4,), jnp.float32),
        ],
        scratch_types=[
            pltpu.VMEM((_HALF,), jnp.int32),
            pltpu.VMEM((_HALF,), jnp.float32),
            pltpu.VMEM((2 * _HALF,), jnp.int32),
            pltpu.VMEM((2 * _HALF,), jnp.float32),
            pltpu.VMEM((2 * _HALF,), jnp.float32),
            pltpu.VMEM((_HALF,), jnp.float32),
            pltpu.VMEM((_HALF,), jnp.float32),
            pltpu.VMEM((_HALF,), jnp.float32),
            pltpu.VMEM((_HALF,), jnp.float32),
            pltpu.VMEM((_HALF,), jnp.float32),
            pltpu.VMEM((_HALF,), jnp.float32),
            pltpu.SemaphoreType.DMA,
            pltpu.SemaphoreType.DMA,
        ],
    )(_decode_body)(*args)


@jax.jit
def kernel(heatmap, offset, wh):
    hm3 = heatmap.reshape(_B, _H, _W)
    sc, fl = _topk_call(hm3)
    off_rows = offset.reshape(-1)
    wh_rows = wh.reshape(-1)
    ids_p, sco_p, bb_p = _decode_call(
        sc.reshape(-1), fl.reshape(-1), off_rows, wh_rows)
    ids = ids_p.reshape(_B, _KPAD)[:, :_TOPK, None]
    scores = sco_p.reshape(_B, _KPAD)[:, :_TOPK, None]
    bboxes = jnp.transpose(
        bb_p.reshape(4, _B, _KPAD), (1, 2, 0))[:, :_TOPK, :]
    return ids, scores, bboxes


# speedup vs baseline: 11.4653x; 1.0474x over previous
import functools

import jax
import jax.numpy as jnp
from jax import lax
from jax.experimental import pallas as pl
from jax.experimental.pallas import tpu as pltpu
from jax.experimental.pallas import tpu_sc as plsc

_B = 16
_H = 256
_W = 256
_TOPK = 100
_KPAD = 128
_SCALE = 4.0
_THRESH = 0.01
_BIG = 1 << 30


def _topk_body(hm_ref, sc_ref, fl_ref, rows_ref, *mask_refs):
    h = hm_ref[...]
    ninf = jnp.float32(-jnp.inf)
    col_pad = jnp.full((_B, _H, 1), ninf, jnp.float32)
    row_pad = jnp.full((_B, 1, _W), ninf, jnp.float32)
    left = jnp.concatenate([col_pad, h[:, :, :-1]], axis=2)
    right = jnp.concatenate([h[:, :, 1:], col_pad], axis=2)
    cm = jnp.maximum(jnp.maximum(left, right), h)
    up = jnp.concatenate([row_pad, cm[:, :-1, :]], axis=1)
    down = jnp.concatenate([cm[:, 1:, :], row_pad], axis=1)
    pm = jnp.maximum(jnp.maximum(up, down), cm)
    masked = jnp.where(pm == h, h, jnp.float32(0.0))
    for b in range(_B):
        mask_refs[b][...] = masked[b]

    rowmax = jnp.max(masked, axis=2)
    iota_j3 = lax.broadcasted_iota(jnp.int32, (_B, _H, _W), 2)
    rowarg = jnp.min(
        jnp.where(masked == rowmax[:, :, None], iota_j3, _BIG), axis=2)

    lane_i = lax.broadcasted_iota(jnp.int32, (_B, _H), 1)
    batch_16 = lax.broadcasted_iota(jnp.int32, (_B, 1), 0)
    lane_k = lax.broadcasted_iota(jnp.int32, (_B, _KPAD), 1)
    lane_1 = lax.broadcasted_iota(jnp.int32, (1, _W), 1)

    def step(k, carry):
        rowmax, rowarg, sc_v, fl_v = carry
        gm = jnp.max(rowmax, axis=1, keepdims=True)
        cand = jnp.where(rowmax == gm, lane_i * _W + rowarg, _BIG)
        flat_v = jnp.min(cand, axis=1, keepdims=True)
        nrs = []
        for b in range(_B):
            fb = jnp.min(jnp.where(batch_16 == b, flat_v, _BIG))
            ib = fb // _W
            jb = fb % _W
            row = mask_refs[b][pl.ds(ib, 1), :]
            nr = jnp.where(lane_1 == jb, jnp.float32(-1.0), row)
            mask_refs[b][pl.ds(ib, 1), :] = nr
            nrs.append(nr)
        newrows = jnp.concatenate(nrs, axis=0)
        nm = jnp.max(newrows, axis=1, keepdims=True)
        na = jnp.min(jnp.where(newrows == nm, lane_i, _BIG),
                     axis=1, keepdims=True)
        cond = lane_i == flat_v // _W
        rowmax = jnp.where(cond, nm, rowmax)
        rowarg = jnp.where(cond, na, rowarg)
        ksel = lane_k == k
        sc_v = jnp.where(ksel, gm, sc_v)
        fl_v = jnp.where(ksel, flat_v, fl_v)
        return rowmax, rowarg, sc_v, fl_v

    init = (rowmax, rowarg, jnp.zeros((_B, _KPAD), jnp.float32),
            jnp.zeros((_B, _KPAD), jnp.int32))
    _, _, sc_v, fl_v = lax.fori_loop(0, _TOPK, step, init, unroll=2)
    sc_ref[...] = sc_v
    fl_ref[...] = fl_v


def _topk_call(hm3):
    return pl.pallas_call(
        _topk_body,
        in_specs=[pl.BlockSpec((_B, _H, _W), lambda: (0, 0, 0))],
        out_specs=[
            pl.BlockSpec((_B, _KPAD), lambda: (0, 0)),
            pl.BlockSpec((_B, _KPAD), lambda: (0, 0)),
        ],
        out_shape=[
            jax.ShapeDtypeStruct((_B, _KPAD), jnp.float32),
            jax.ShapeDtypeStruct((_B, _KPAD), jnp.int32),
        ],
        scratch_shapes=[pltpu.VMEM((_B, _W), jnp.float32)] + [
            pltpu.VMEM((_H, _W), jnp.float32) for _ in range(_B)],
    )(hm3)


_HALF = 64


def _decode_body(sc_hbm, fl_hbm, off_hbm, wh_hbm, ids_out, sco_out, bb_out,
                 sidx, sscore, pobuf, obuf, wbuf, idsb, scb,
                 x1b, y1b, x2b, y2b, sem1, sem2):
    wid = lax.axis_index("s") * 2 + lax.axis_index("c")
    b = wid // 2
    start = (wid % 2) * _HALF
    pltpu.sync_copy(fl_hbm.at[pl.ds(b * _KPAD + start, _HALF)], sidx)
    pltpu.sync_copy(sc_hbm.at[pl.ds(b * _KPAD + start, _HALF)], sscore)
    base = b * (2 * _H * _W)
    for k in range(_HALF // 16):
        idx = sidx[pl.ds(k * 16, 16)]
        pobuf[pl.ds(k * 16, 16)] = base + idx
        pobuf[pl.ds(_HALF + k * 16, 16)] = base + _H * _W + idx
    pltpu.async_copy(off_hbm.at[pobuf], obuf, sem1).wait()
    pltpu.async_copy(wh_hbm.at[pobuf], wbuf, sem2).wait()
    lane = lax.iota(jnp.int32, 16)
    neg1 = jnp.float32(-1.0)
    for k in range(_HALF // 16):
        idx = sidx[pl.ds(k * 16, 16)]
        s = sscore[pl.ds(k * 16, 16)]
        y = lax.shift_right_logical(idx, 8)
        x = jnp.bitwise_and(idx, _W - 1)
        ox = obuf[pl.ds(k * 16, 16)]
        oy = obuf[pl.ds(_HALF + k * 16, 16)]
        ww = wbuf[pl.ds(k * 16, 16)]
        hh = wbuf[pl.ds(_HALF + k * 16, 16)]
        keep = s > _THRESH
        cx = x.astype(jnp.float32) + ox
        cy = y.astype(jnp.float32) + oy
        hw = ww * 0.5
        hh2 = hh * 0.5
        x1 = jnp.where(keep, cx - hw, neg1) * _SCALE
        y1 = jnp.where(keep, cy - hh2, neg1) * _SCALE
        x2 = jnp.where(keep, cx + hw, neg1) * _SCALE
        y2 = jnp.where(keep, cy + hh2, neg1) * _SCALE
        idsb[pl.ds(k * 16, 16)] = jnp.where(keep, jnp.float32(0.0), neg1)
        scb[pl.ds(k * 16, 16)] = jnp.where(keep, s, neg1)
        x1b[pl.ds(k * 16, 16)] = x1
        y1b[pl.ds(k * 16, 16)] = y1
        x2b[pl.ds(k * 16, 16)] = x2
        y2b[pl.ds(k * 16, 16)] = y2
    pos = b * _KPAD + start
    pltpu.sync_copy(idsb, ids_out.at[pl.ds(pos, _HALF)])
    pltpu.sync_copy(scb, sco_out.at[pl.ds(pos, _HALF)])
    n = _B * _KPAD
    pltpu.sync_copy(x1b, bb_out.at[pl.ds(pos, _HALF)])
    pltpu.sync_copy(y1b, bb_out.at[pl.ds(n + pos, _HALF)])
    pltpu.sync_copy(x2b, bb_out.at[pl.ds(2 * n + pos, _HALF)])
    pltpu.sync_copy(y2b, bb_out.at[pl.ds(3 * n + pos, _HALF)])


def _decode_call(*args):
    return functools.partial(
        pl.kernel,
        mesh=plsc.VectorSubcoreMesh(core_axis_name="c", subcore_axis_name="s"),
        compiler_params=pltpu.CompilerParams(use_tc_tiling_on_sc=False),
        out_type=[
            jax.ShapeDtypeStruct((_B * _KPAD,), jnp.float32),
            jax.ShapeDtypeStruct((_B * _KPAD,), jnp.float32),
            jax.ShapeDtypeStruct((_B * _KPAD * 4,), jnp.float32),
        ],
        scratch_types=[
            pltpu.VMEM((_HALF,), jnp.int32),
            pltpu.VMEM((_HALF,), jnp.float32),
            pltpu.VMEM((2 * _HALF,), jnp.int32),
            pltpu.VMEM((2 * _HALF,), jnp.float32),
            pltpu.VMEM((2 * _HALF,), jnp.float32),
            pltpu.VMEM((_HALF,), jnp.float32),
            pltpu.VMEM((_HALF,), jnp.float32),
            pltpu.VMEM((_HALF,), jnp.float32),
            pltpu.VMEM((_HALF,), jnp.float32),
            pltpu.VMEM((_HALF,), jnp.float32),
            pltpu.VMEM((_HALF,), jnp.float32),
            pltpu.SemaphoreType.DMA,
            pltpu.SemaphoreType.DMA,
        ],
    )(_decode_body)(*args)


@jax.jit
def kernel(heatmap, offset, wh):
    hm3 = heatmap.reshape(_B, _H, _W)
    sc, fl = _topk_call(hm3)
    off_rows = offset.reshape(-1)
    wh_rows = wh.reshape(-1)
    ids_p, sco_p, bb_p = _decode_call(
        sc.reshape(-1), fl.reshape(-1), off_rows, wh_rows)
    ids = ids_p.reshape(_B, _KPAD)[:, :_TOPK, None]
    scores = sco_p.reshape(_B, _KPAD)[:, :_TOPK, None]
    bboxes = jnp.transpose(
        bb_p.reshape(4, _B, _KPAD), (1, 2, 0))[:, :_TOPK, :]
    return ids, scores, bboxes


# speedup vs baseline: 11.6707x; 1.0179x over previous
import functools

import jax
import jax.numpy as jnp
from jax import lax
from jax.experimental import pallas as pl
from jax.experimental.pallas import tpu as pltpu
from jax.experimental.pallas import tpu_sc as plsc

_B = 16
_H = 256
_W = 256
_TOPK = 100
_KPAD = 128
_SCALE = 4.0
_THRESH = 0.01
_BIG = 1 << 30


def _topk_body(hm_ref, sc_ref, fl_ref, rows_ref, *mask_refs):
    h = hm_ref[...]
    ninf = jnp.float32(-jnp.inf)
    col_pad = jnp.full((_B, _H, 1), ninf, jnp.float32)
    row_pad = jnp.full((_B, 1, _W), ninf, jnp.float32)
    left = jnp.concatenate([col_pad, h[:, :, :-1]], axis=2)
    right = jnp.concatenate([h[:, :, 1:], col_pad], axis=2)
    cm = jnp.maximum(jnp.maximum(left, right), h)
    up = jnp.concatenate([row_pad, cm[:, :-1, :]], axis=1)
    down = jnp.concatenate([cm[:, 1:, :], row_pad], axis=1)
    pm = jnp.maximum(jnp.maximum(up, down), cm)
    masked = jnp.where(pm == h, h, jnp.float32(0.0))
    for b in range(_B):
        mask_refs[b][...] = masked[b]

    rowmax = jnp.max(masked, axis=2)
    iota_j3 = lax.broadcasted_iota(jnp.int32, (_B, _H, _W), 2)
    rowarg = jnp.min(
        jnp.where(masked == rowmax[:, :, None], iota_j3, _BIG), axis=2)

    lane_i = lax.broadcasted_iota(jnp.int32, (_B, _H), 1)
    batch_16 = lax.broadcasted_iota(jnp.int32, (_B, 1), 0)
    lane_k = lax.broadcasted_iota(jnp.int32, (_B, _KPAD), 1)
    lane_1 = lax.broadcasted_iota(jnp.int32, (1, _W), 1)

    def step(k, carry):
        rowmax, rowarg, sc_v, fl_v = carry
        gm = jnp.max(rowmax, axis=1, keepdims=True)
        cand = jnp.where(rowmax == gm, lane_i * _W + rowarg, _BIG)
        flat_v = jnp.min(cand, axis=1, keepdims=True)
        nrs = []
        for b in range(_B):
            fb = jnp.min(jnp.where(batch_16 == b, flat_v, _BIG))
            ib = fb // _W
            jb = fb % _W
            row = mask_refs[b][pl.ds(ib, 1), :]
            nr = jnp.where(lane_1 == jb, jnp.float32(-1.0), row)
            mask_refs[b][pl.ds(ib, 1), :] = nr
            nrs.append(nr)
        newrows = jnp.concatenate(nrs, axis=0)
        nm = jnp.max(newrows, axis=1, keepdims=True)
        na = jnp.min(jnp.where(newrows == nm, lane_i, _BIG),
                     axis=1, keepdims=True)
        cond = lane_i == flat_v // _W
        rowmax = jnp.where(cond, nm, rowmax)
        rowarg = jnp.where(cond, na, rowarg)
        ksel = lane_k == k
        sc_v = jnp.where(ksel, gm, sc_v)
        fl_v = jnp.where(ksel, flat_v, fl_v)
        return rowmax, rowarg, sc_v, fl_v

    init = (rowmax, rowarg, jnp.zeros((_B, _KPAD), jnp.float32),
            jnp.zeros((_B, _KPAD), jnp.int32))
    _, _, sc_v, fl_v = lax.fori_loop(0, _TOPK, step, init, unroll=4)
    sc_ref[...] = sc_v
    fl_ref[...] = fl_v


def _topk_call(hm3):
    return pl.pallas_call(
        _topk_body,
        in_specs=[pl.BlockSpec((_B, _H, _W), lambda: (0, 0, 0))],
        out_specs=[
            pl.BlockSpec((_B, _KPAD), lambda: (0, 0)),
            pl.BlockSpec((_B, _KPAD), lambda: (0, 0)),
        ],
        out_shape=[
            jax.ShapeDtypeStruct((_B, _KPAD), jnp.float32),
            jax.ShapeDtypeStruct((_B, _KPAD), jnp.int32),
        ],
        scratch_shapes=[pltpu.VMEM((_B, _W), jnp.float32)] + [
            pltpu.VMEM((_H, _W), jnp.float32) for _ in range(_B)],
    )(hm3)


_HALF = 64


def _decode_body(sc_hbm, fl_hbm, off_hbm, wh_hbm, ids_out, sco_out, bb_out,
                 sidx, sscore, pobuf, obuf, wbuf, idsb, scb,
                 x1b, y1b, x2b, y2b, sem1, sem2):
    wid = lax.axis_index("s") * 2 + lax.axis_index("c")
    b = wid // 2
    start = (wid % 2) * _HALF
    pltpu.sync_copy(fl_hbm.at[pl.ds(b * _KPAD + start, _HALF)], sidx)
    pltpu.sync_copy(sc_hbm.at[pl.ds(b * _KPAD + start, _HALF)], sscore)
    base = b * (2 * _H * _W)
    for k in range(_HALF // 16):
        idx = sidx[pl.ds(k * 16, 16)]
        pobuf[pl.ds(k * 16, 16)] = base + idx
        pobuf[pl.ds(_HALF + k * 16, 16)] = base + _H * _W + idx
    pltpu.async_copy(off_hbm.at[pobuf], obuf, sem1).wait()
    pltpu.async_copy(wh_hbm.at[pobuf], wbuf, sem2).wait()
    lane = lax.iota(jnp.int32, 16)
    neg1 = jnp.float32(-1.0)
    for k in range(_HALF // 16):
        idx = sidx[pl.ds(k * 16, 16)]
        s = sscore[pl.ds(k * 16, 16)]
        y = lax.shift_right_logical(idx, 8)
        x = jnp.bitwise_and(idx, _W - 1)
        ox = obuf[pl.ds(k * 16, 16)]
        oy = obuf[pl.ds(_HALF + k * 16, 16)]
        ww = wbuf[pl.ds(k * 16, 16)]
        hh = wbuf[pl.ds(_HALF + k * 16, 16)]
        keep = s > _THRESH
        cx = x.astype(jnp.float32) + ox
        cy = y.astype(jnp.float32) + oy
        hw = ww * 0.5
        hh2 = hh * 0.5
        x1 = jnp.where(keep, cx - hw, neg1) * _SCALE
        y1 = jnp.where(keep, cy - hh2, neg1) * _SCALE
        x2 = jnp.where(keep, cx + hw, neg1) * _SCALE
        y2 = jnp.where(keep, cy + hh2, neg1) * _SCALE
        idsb[pl.ds(k * 16, 16)] = jnp.where(keep, jnp.float32(0.0), neg1)
        scb[pl.ds(k * 16, 16)] = jnp.where(keep, s, neg1)
        x1b[pl.ds(k * 16, 16)] = x1
        y1b[pl.ds(k * 16, 16)] = y1
        x2b[pl.ds(k * 16, 16)] = x2
        y2b[pl.ds(k * 16, 16)] = y2
    pos = b * _KPAD + start
    pltpu.sync_copy(idsb, ids_out.at[pl.ds(pos, _HALF)])
    pltpu.sync_copy(scb, sco_out.at[pl.ds(pos, _HALF)])
    n = _B * _KPAD
    pltpu.sync_copy(x1b, bb_out.at[pl.ds(pos, _HALF)])
    pltpu.sync_copy(y1b, bb_out.at[pl.ds(n + pos, _HALF)])
    pltpu.sync_copy(x2b, bb_out.at[pl.ds(2 * n + pos, _HALF)])
    pltpu.sync_copy(y2b, bb_out.at[pl.ds(3 * n + pos, _HALF)])


def _decode_call(*args):
    return functools.partial(
        pl.kernel,
        mesh=plsc.VectorSubcoreMesh(core_axis_name="c", subcore_axis_name="s"),
        compiler_params=pltpu.CompilerParams(use_tc_tiling_on_sc=False),
        out_type=[
            jax.ShapeDtypeStruct((_B * _KPAD,), jnp.float32),
            jax.ShapeDtypeStruct((_B * _KPAD,), jnp.float32),
            jax.ShapeDtypeStruct((_B * _KPAD * 4,), jnp.float32),
        ],
        scratch_types=[
            pltpu.VMEM((_HALF,), jnp.int32),
            pltpu.VMEM((_HALF,), jnp.float32),
            pltpu.VMEM((2 * _HALF,), jnp.int32),
            pltpu.VMEM((2 * _HALF,), jnp.float32),
            pltpu.VMEM((2 * _HALF,), jnp.float32),
            pltpu.VMEM((_HALF,), jnp.float32),
            pltpu.VMEM((_HALF,), jnp.float32),
            pltpu.VMEM((_HALF,), jnp.float32),
            pltpu.VMEM((_HALF,), jnp.float32),
            pltpu.VMEM((_HALF,), jnp.float32),
            pltpu.VMEM((_HALF,), jnp.float32),
            pltpu.SemaphoreType.DMA,
            pltpu.SemaphoreType.DMA,
        ],
    )(_decode_body)(*args)


@jax.jit
def kernel(heatmap, offset, wh):
    hm3 = heatmap.reshape(_B, _H, _W)
    sc, fl = _topk_call(hm3)
    off_rows = offset.reshape(-1)
    wh_rows = wh.reshape(-1)
    ids_p, sco_p, bb_p = _decode_call(
        sc.reshape(-1), fl.reshape(-1), off_rows, wh_rows)
    ids = ids_p.reshape(_B, _KPAD)[:, :_TOPK, None]
    scores = sco_p.reshape(_B, _KPAD)[:, :_TOPK, None]
    bboxes = jnp.transpose(
        bb_p.reshape(4, _B, _KPAD), (1, 2, 0))[:, :_TOPK, :]
    return ids, scores, bboxes


# speedup vs baseline: 11.8105x; 1.0120x over previous
import functools

import jax
import jax.numpy as jnp
from jax import lax
from jax.experimental import pallas as pl
from jax.experimental.pallas import tpu as pltpu
from jax.experimental.pallas import tpu_sc as plsc

_B = 16
_H = 256
_W = 256
_TOPK = 100
_KPAD = 128
_SCALE = 4.0
_THRESH = 0.01
_BIG = 1 << 30


def _topk_body(hm_ref, sc_ref, fl_ref, rows_ref, *mask_refs):
    h = hm_ref[...]
    ninf = jnp.float32(-jnp.inf)
    col_pad = jnp.full((_B, _H, 1), ninf, jnp.float32)
    row_pad = jnp.full((_B, 1, _W), ninf, jnp.float32)
    left = jnp.concatenate([col_pad, h[:, :, :-1]], axis=2)
    right = jnp.concatenate([h[:, :, 1:], col_pad], axis=2)
    cm = jnp.maximum(jnp.maximum(left, right), h)
    up = jnp.concatenate([row_pad, cm[:, :-1, :]], axis=1)
    down = jnp.concatenate([cm[:, 1:, :], row_pad], axis=1)
    pm = jnp.maximum(jnp.maximum(up, down), cm)
    masked = jnp.where(pm == h, h, jnp.float32(0.0))
    for b in range(_B):
        mask_refs[b][...] = masked[b]

    rowmax = jnp.max(masked, axis=2)
    iota_j3 = lax.broadcasted_iota(jnp.int32, (_B, _H, _W), 2)
    rowarg = jnp.min(
        jnp.where(masked == rowmax[:, :, None], iota_j3, _BIG), axis=2)

    lane_i = lax.broadcasted_iota(jnp.int32, (_B, _H), 1)
    batch_16 = lax.broadcasted_iota(jnp.int32, (_B, 1), 0)
    lane_k = lax.broadcasted_iota(jnp.int32, (_B, _KPAD), 1)
    lane_1 = lax.broadcasted_iota(jnp.int32, (1, _W), 1)

    def step(k, carry):
        rowmax, rowarg, sc_v, fl_v = carry
        gm = jnp.max(rowmax, axis=1, keepdims=True)
        cand = jnp.where(rowmax == gm, lane_i * _W + rowarg, _BIG)
        flat_v = jnp.min(cand, axis=1, keepdims=True)
        nrs = []
        for b in range(_B):
            fb = jnp.min(jnp.where(batch_16 == b, flat_v, _BIG))
            ib = fb // _W
            jb = fb % _W
            row = mask_refs[b][pl.ds(ib, 1), :]
            nr = jnp.where(lane_1 == jb, jnp.float32(-1.0), row)
            mask_refs[b][pl.ds(ib, 1), :] = nr
            nrs.append(nr)
        newrows = jnp.concatenate(nrs, axis=0)
        nm = jnp.max(newrows, axis=1, keepdims=True)
        na = jnp.min(jnp.where(newrows == nm, lane_i, _BIG),
                     axis=1, keepdims=True)
        cond = lane_i == flat_v // _W
        rowmax = jnp.where(cond, nm, rowmax)
        rowarg = jnp.where(cond, na, rowarg)
        ksel = lane_k == k
        sc_v = jnp.where(ksel, gm, sc_v)
        fl_v = jnp.where(ksel, flat_v, fl_v)
        return rowmax, rowarg, sc_v, fl_v

    init = (rowmax, rowarg, jnp.zeros((_B, _KPAD), jnp.float32),
            jnp.zeros((_B, _KPAD), jnp.int32))
    _, _, sc_v, fl_v = lax.fori_loop(0, _TOPK, step, init, unroll=8)
    sc_ref[...] = sc_v
    fl_ref[...] = fl_v


def _topk_call(hm3):
    return pl.pallas_call(
        _topk_body,
        in_specs=[pl.BlockSpec((_B, _H, _W), lambda: (0, 0, 0))],
        out_specs=[
            pl.BlockSpec((_B, _KPAD), lambda: (0, 0)),
            pl.BlockSpec((_B, _KPAD), lambda: (0, 0)),
        ],
        out_shape=[
            jax.ShapeDtypeStruct((_B, _KPAD), jnp.float32),
            jax.ShapeDtypeStruct((_B, _KPAD), jnp.int32),
        ],
        scratch_shapes=[pltpu.VMEM((_B, _W), jnp.float32)] + [
            pltpu.VMEM((_H, _W), jnp.float32) for _ in range(_B)],
    )(hm3)


_HALF = 64


def _decode_body(sc_hbm, fl_hbm, off_hbm, wh_hbm, ids_out, sco_out, bb_out,
                 sidx, sscore, pobuf, obuf, wbuf, idsb, scb,
                 x1b, y1b, x2b, y2b, sem1, sem2):
    wid = lax.axis_index("s") * 2 + lax.axis_index("c")
    b = wid // 2
    start = (wid % 2) * _HALF
    pltpu.sync_copy(fl_hbm.at[pl.ds(b * _KPAD + start, _HALF)], sidx)
    pltpu.sync_copy(sc_hbm.at[pl.ds(b * _KPAD + start, _HALF)], sscore)
    base = b * (2 * _H * _W)
    for k in range(_HALF // 16):
        idx = sidx[pl.ds(k * 16, 16)]
        pobuf[pl.ds(k * 16, 16)] = base + idx
        pobuf[pl.ds(_HALF + k * 16, 16)] = base + _H * _W + idx
    pltpu.async_copy(off_hbm.at[pobuf], obuf, sem1).wait()
    pltpu.async_copy(wh_hbm.at[pobuf], wbuf, sem2).wait()
    lane = lax.iota(jnp.int32, 16)
    neg1 = jnp.float32(-1.0)
    for k in range(_HALF // 16):
        idx = sidx[pl.ds(k * 16, 16)]
        s = sscore[pl.ds(k * 16, 16)]
        y = lax.shift_right_logical(idx, 8)
        x = jnp.bitwise_and(idx, _W - 1)
        ox = obuf[pl.ds(k * 16, 16)]
        oy = obuf[pl.ds(_HALF + k * 16, 16)]
        ww = wbuf[pl.ds(k * 16, 16)]
        hh = wbuf[pl.ds(_HALF + k * 16, 16)]
        keep = s > _THRESH
        cx = x.astype(jnp.float32) + ox
        cy = y.astype(jnp.float32) + oy
        hw = ww * 0.5
        hh2 = hh * 0.5
        x1 = jnp.where(keep, cx - hw, neg1) * _SCALE
        y1 = jnp.where(keep, cy - hh2, neg1) * _SCALE
        x2 = jnp.where(keep, cx + hw, neg1) * _SCALE
        y2 = jnp.where(keep, cy + hh2, neg1) * _SCALE
        idsb[pl.ds(k * 16, 16)] = jnp.where(keep, jnp.float32(0.0), neg1)
        scb[pl.ds(k * 16, 16)] = jnp.where(keep, s, neg1)
        x1b[pl.ds(k * 16, 16)] = x1
        y1b[pl.ds(k * 16, 16)] = y1
        x2b[pl.ds(k * 16, 16)] = x2
        y2b[pl.ds(k * 16, 16)] = y2
    pos = b * _KPAD + start
    pltpu.sync_copy(idsb, ids_out.at[pl.ds(pos, _HALF)])
    pltpu.sync_copy(scb, sco_out.at[pl.ds(pos, _HALF)])
    n = _B * _KPAD
    pltpu.sync_copy(x1b, bb_out.at[pl.ds(pos, _HALF)])
    pltpu.sync_copy(y1b, bb_out.at[pl.ds(n + pos, _HALF)])
    pltpu.sync_copy(x2b, bb_out.at[pl.ds(2 * n + pos, _HALF)])
    pltpu.sync_copy(y2b, bb_out.at[pl.ds(3 * n + pos, _HALF)])


def _decode_call(*args):
    return functools.partial(
        pl.kernel,
        mesh=plsc.VectorSubcoreMesh(core_axis_name="c", subcore_axis_name="s"),
        compiler_params=pltpu.CompilerParams(use_tc_tiling_on_sc=False),
        out_type=[
            jax.ShapeDtypeStruct((_B * _KPAD,), jnp.float32),
            jax.ShapeDtypeStruct((_B * _KPAD,), jnp.float32),
            jax.ShapeDtypeStruct((_B * _KPAD * 4,), jnp.float32),
        ],
        scratch_types=[
            pltpu.VMEM((_HALF,), jnp.int32),
            pltpu.VMEM((_HALF,), jnp.float32),
            pltpu.VMEM((2 * _HALF,), jnp.int32),
            pltpu.VMEM((2 * _HALF,), jnp.float32),
            pltpu.VMEM((2 * _HALF,), jnp.float32),
            pltpu.VMEM((_HALF,), jnp.float32),
            pltpu.VMEM((_HALF,), jnp.float32),
            pltpu.VMEM((_HALF,), jnp.float32),
            pltpu.VMEM((_HALF,), jnp.float32),
            pltpu.VMEM((_HALF,), jnp.float32),
            pltpu.VMEM((_HALF,), jnp.float32),
            pltpu.SemaphoreType.DMA,
            pltpu.SemaphoreType.DMA,
        ],
    )(_decode_body)(*args)


@jax.jit
def kernel(heatmap, offset, wh):
    hm3 = heatmap.reshape(_B, _H, _W)
    sc, fl = _topk_call(hm3)
    off_rows = offset.reshape(-1)
    wh_rows = wh.reshape(-1)
    ids_p, sco_p, bb_p = _decode_call(
        sc.reshape(-1), fl.reshape(-1), off_rows, wh_rows)
    ids = ids_p.reshape(_B, _KPAD)[:, :_TOPK, None]
    scores = sco_p.reshape(_B, _KPAD)[:, :_TOPK, None]
    bboxes = jnp.transpose(
        bb_p.reshape(4, _B, _KPAD), (1, 2, 0))[:, :_TOPK, :]
    return ids, scores, bboxes


# speedup vs baseline: 11.8131x; 1.0002x over previous
import functools

import jax
import jax.numpy as jnp
from jax import lax
from jax.experimental import pallas as pl
from jax.experimental.pallas import tpu as pltpu
from jax.experimental.pallas import tpu_sc as plsc

_B = 16
_H = 256
_W = 256
_TOPK = 100
_KPAD = 128
_SCALE = 4.0
_THRESH = 0.01
_BIG = 1 << 30


def _topk_body(hm_ref, sc_ref, fl_ref, *mask_refs):
    h = hm_ref[...]
    ninf = jnp.float32(-jnp.inf)
    col_pad = jnp.full((_B, _H, 1), ninf, jnp.float32)
    row_pad = jnp.full((_B, 1, _W), ninf, jnp.float32)
    left = jnp.concatenate([col_pad, h[:, :, :-1]], axis=2)
    right = jnp.concatenate([h[:, :, 1:], col_pad], axis=2)
    cm = jnp.maximum(jnp.maximum(left, right), h)
    up = jnp.concatenate([row_pad, cm[:, :-1, :]], axis=1)
    down = jnp.concatenate([cm[:, 1:, :], row_pad], axis=1)
    pm = jnp.maximum(jnp.maximum(up, down), cm)
    masked = jnp.where(pm == h, h, jnp.float32(0.0))
    for b in range(_B):
        mask_refs[b][...] = masked[b]

    rowmax = jnp.max(masked, axis=2)
    iota_j3 = lax.broadcasted_iota(jnp.int32, (_B, _H, _W), 2)
    rowarg = jnp.min(
        jnp.where(masked == rowmax[:, :, None], iota_j3, _BIG), axis=2)

    lane_i = lax.broadcasted_iota(jnp.int32, (_B, _H), 1)
    batch_16 = lax.broadcasted_iota(jnp.int32, (_B, 1), 0)
    lane_k = lax.broadcasted_iota(jnp.int32, (_B, _KPAD), 1)
    lane_1 = lax.broadcasted_iota(jnp.int32, (1, _W), 1)

    def step(k, carry):
        rowmax, rowarg, sc_v, fl_v = carry
        gm = jnp.max(rowmax, axis=1, keepdims=True)
        cand = jnp.where(rowmax == gm, lane_i * _W + rowarg, _BIG)
        flat_v = jnp.min(cand, axis=1, keepdims=True)
        nrs = []
        for b in range(_B):
            fb = jnp.min(jnp.where(batch_16 == b, flat_v, _BIG))
            ib = fb // _W
            jb = fb % _W
            row = mask_refs[b][pl.ds(ib, 1), :]
            nr = jnp.where(lane_1 == jb, jnp.float32(-1.0), row)
            mask_refs[b][pl.ds(ib, 1), :] = nr
            nrs.append(nr)
        newrows = jnp.concatenate(nrs, axis=0)
        nm = jnp.max(newrows, axis=1, keepdims=True)
        na = jnp.min(jnp.where(newrows == nm, lane_i, _BIG),
                     axis=1, keepdims=True)
        cond = lane_i == flat_v // _W
        rowmax = jnp.where(cond, nm, rowmax)
        rowarg = jnp.where(cond, na, rowarg)
        ksel = lane_k == k
        sc_v = jnp.where(ksel, gm, sc_v)
        fl_v = jnp.where(ksel, flat_v, fl_v)
        return rowmax, rowarg, sc_v, fl_v

    init = (rowmax, rowarg, jnp.zeros((_B, _KPAD), jnp.float32),
            jnp.zeros((_B, _KPAD), jnp.int32))
    _, _, sc_v, fl_v = lax.fori_loop(0, _TOPK, step, init, unroll=8)
    sc_ref[...] = sc_v
    fl_ref[...] = fl_v


def _topk_call(hm3):
    return pl.pallas_call(
        _topk_body,
        in_specs=[pl.BlockSpec((_B, _H, _W), lambda: (0, 0, 0))],
        out_specs=[
            pl.BlockSpec((_B, _KPAD), lambda: (0, 0)),
            pl.BlockSpec((_B, _KPAD), lambda: (0, 0)),
        ],
        out_shape=[
            jax.ShapeDtypeStruct((_B, _KPAD), jnp.float32),
            jax.ShapeDtypeStruct((_B, _KPAD), jnp.int32),
        ],
        scratch_shapes=[
            pltpu.VMEM((_H, _W), jnp.float32) for _ in range(_B)],
    )(hm3)


_HALF = 64


def _decode_body(sc_hbm, fl_hbm, off_hbm, wh_hbm, ids_out, sco_out, bb_out,
                 sidx, sscore, pobuf, obuf, wbuf, idsb, scb,
                 x1b, y1b, x2b, y2b, sem1, sem2):
    wid = lax.axis_index("s") * 2 + lax.axis_index("c")
    b = wid // 2
    start = (wid % 2) * _HALF
    pltpu.sync_copy(fl_hbm.at[pl.ds(b * _KPAD + start, _HALF)], sidx)
    pltpu.sync_copy(sc_hbm.at[pl.ds(b * _KPAD + start, _HALF)], sscore)
    base = b * (2 * _H * _W)
    for k in range(_HALF // 16):
        idx = sidx[pl.ds(k * 16, 16)]
        pobuf[pl.ds(k * 16, 16)] = base + idx
        pobuf[pl.ds(_HALF + k * 16, 16)] = base + _H * _W + idx
    pltpu.async_copy(off_hbm.at[pobuf], obuf, sem1).wait()
    pltpu.async_copy(wh_hbm.at[pobuf], wbuf, sem2).wait()
    neg1 = jnp.float32(-1.0)
    for k in range(_HALF // 16):
        idx = sidx[pl.ds(k * 16, 16)]
        s = sscore[pl.ds(k * 16, 16)]
        y = lax.shift_right_logical(idx, 8)
        x = jnp.bitwise_and(idx, _W - 1)
        ox = obuf[pl.ds(k * 16, 16)]
        oy = obuf[pl.ds(_HALF + k * 16, 16)]
        ww = wbuf[pl.ds(k * 16, 16)]
        hh = wbuf[pl.ds(_HALF + k * 16, 16)]
        keep = s > _THRESH
        cx = x.astype(jnp.float32) + ox
        cy = y.astype(jnp.float32) + oy
        hw = ww * 0.5
        hh2 = hh * 0.5
        x1 = jnp.where(keep, cx - hw, neg1) * _SCALE
        y1 = jnp.where(keep, cy - hh2, neg1) * _SCALE
        x2 = jnp.where(keep, cx + hw, neg1) * _SCALE
        y2 = jnp.where(keep, cy + hh2, neg1) * _SCALE
        idsb[pl.ds(k * 16, 16)] = jnp.where(keep, jnp.float32(0.0), neg1)
        scb[pl.ds(k * 16, 16)] = jnp.where(keep, s, neg1)
        x1b[pl.ds(k * 16, 16)] = x1
        y1b[pl.ds(k * 16, 16)] = y1
        x2b[pl.ds(k * 16, 16)] = x2
        y2b[pl.ds(k * 16, 16)] = y2
    pos = b * _KPAD + start
    pltpu.sync_copy(idsb, ids_out.at[pl.ds(pos, _HALF)])
    pltpu.sync_copy(scb, sco_out.at[pl.ds(pos, _HALF)])
    n = _B * _KPAD
    pltpu.sync_copy(x1b, bb_out.at[pl.ds(pos, _HALF)])
    pltpu.sync_copy(y1b, bb_out.at[pl.ds(n + pos, _HALF)])
    pltpu.sync_copy(x2b, bb_out.at[pl.ds(2 * n + pos, _HALF)])
    pltpu.sync_copy(y2b, bb_out.at[pl.ds(3 * n + pos, _HALF)])


def _decode_call(*args):
    return functools.partial(
        pl.kernel,
        mesh=plsc.VectorSubcoreMesh(core_axis_name="c", subcore_axis_name="s"),
        compiler_params=pltpu.CompilerParams(use_tc_tiling_on_sc=False),
        out_type=[
            jax.ShapeDtypeStruct((_B * _KPAD,), jnp.float32),
            jax.ShapeDtypeStruct((_B * _KPAD,), jnp.float32),
            jax.ShapeDtypeStruct((_B * _KPAD * 4,), jnp.float32),
        ],
        scratch_types=[
            pltpu.VMEM((_HALF,), jnp.int32),
            pltpu.VMEM((_HALF,), jnp.float32),
            pltpu.VMEM((2 * _HALF,), jnp.int32),
            pltpu.VMEM((2 * _HALF,), jnp.float32),
            pltpu.VMEM((2 * _HALF,), jnp.float32),
            pltpu.VMEM((_HALF,), jnp.float32),
            pltpu.VMEM((_HALF,), jnp.float32),
            pltpu.VMEM((_HALF,), jnp.float32),
            pltpu.VMEM((_HALF,), jnp.float32),
            pltpu.VMEM((_HALF,), jnp.float32),
            pltpu.VMEM((_HALF,), jnp.float32),
            pltpu.SemaphoreType.DMA,
            pltpu.SemaphoreType.DMA,
        ],
    )(_decode_body)(*args)


@jax.jit
def kernel(heatmap, offset, wh):
    hm3 = heatmap.reshape(_B, _H, _W)
    sc, fl = _topk_call(hm3)
    off_rows = offset.reshape(-1)
    wh_rows = wh.reshape(-1)
    ids_p, sco_p, bb_p = _decode_call(
        sc.reshape(-1), fl.reshape(-1), off_rows, wh_rows)
    ids = ids_p.reshape(_B, _KPAD)[:, :_TOPK, None]
    scores = sco_p.reshape(_B, _KPAD)[:, :_TOPK, None]
    bboxes = jnp.transpose(
        bb_p.reshape(4, _B, _KPAD), (1, 2, 0))[:, :_TOPK, :]
    return ids, scores, bboxes


# speedup vs baseline: 11.8803x; 1.0057x over previous
import functools

import jax
import jax.numpy as jnp
from jax import lax
from jax.experimental import pallas as pl
from jax.experimental.pallas import tpu as pltpu
from jax.experimental.pallas import tpu_sc as plsc

_B = 16
_H = 256
_W = 256
_TOPK = 100
_KPAD = 128
_SCALE = 4.0
_THRESH = 0.01
_BIG = 1 << 30


def _topk_body(hm_ref, sc_ref, fl_ref, *mask_refs):
    h = hm_ref[...]
    ninf = jnp.float32(-jnp.inf)
    col_pad = jnp.full((_B, _H, 1), ninf, jnp.float32)
    row_pad = jnp.full((_B, 1, _W), ninf, jnp.float32)
    left = jnp.concatenate([col_pad, h[:, :, :-1]], axis=2)
    right = jnp.concatenate([h[:, :, 1:], col_pad], axis=2)
    cm = jnp.maximum(jnp.maximum(left, right), h)
    up = jnp.concatenate([row_pad, cm[:, :-1, :]], axis=1)
    down = jnp.concatenate([cm[:, 1:, :], row_pad], axis=1)
    pm = jnp.maximum(jnp.maximum(up, down), cm)
    masked = jnp.where(pm == h, h, jnp.float32(0.0))
    for b in range(_B):
        mask_refs[b][...] = masked[b]

    rowmax = jnp.max(masked, axis=2)
    iota_j3 = lax.broadcasted_iota(jnp.int32, (_B, _H, _W), 2)
    rowarg = jnp.min(
        jnp.where(masked == rowmax[:, :, None], iota_j3, _BIG), axis=2)

    lane_i = lax.broadcasted_iota(jnp.int32, (_B, _H), 1)
    batch_16 = lax.broadcasted_iota(jnp.int32, (_B, 1), 0)
    lane_k = lax.broadcasted_iota(jnp.int32, (_B, _KPAD), 1)
    lane_1 = lax.broadcasted_iota(jnp.int32, (1, _W), 1)

    def step(k, carry):
        rowmax, rowarg, sc_v, fl_v = carry
        gm = jnp.max(rowmax, axis=1, keepdims=True)
        cand = jnp.where(rowmax == gm, lane_i * _W + rowarg, _BIG)
        flat_v = jnp.min(cand, axis=1, keepdims=True)
        nrs = []
        for b in range(_B):
            fb = jnp.min(jnp.where(batch_16 == b, flat_v, _BIG))
            ib = fb // _W
            jb = fb % _W
            row = mask_refs[b][pl.ds(ib, 1), :]
            nr = jnp.where(lane_1 == jb, jnp.float32(-1.0), row)
            mask_refs[b][pl.ds(ib, 1), :] = nr
            nrs.append(nr)
        newrows = jnp.concatenate(nrs, axis=0)
        nm = jnp.max(newrows, axis=1, keepdims=True)
        na = jnp.min(jnp.where(newrows == nm, lane_i, _BIG),
                     axis=1, keepdims=True)
        cond = lane_i == flat_v // _W
        rowmax = jnp.where(cond, nm, rowmax)
        rowarg = jnp.where(cond, na, rowarg)
        ksel = lane_k == k
        sc_v = jnp.where(ksel, gm, sc_v)
        fl_v = jnp.where(ksel, flat_v, fl_v)
        return rowmax, rowarg, sc_v, fl_v

    init = (rowmax, rowarg, jnp.zeros((_B, _KPAD), jnp.float32),
            jnp.zeros((_B, _KPAD), jnp.int32))
    _, _, sc_v, fl_v = lax.fori_loop(0, _TOPK, step, init, unroll=16)
    sc_ref[...] = sc_v
    fl_ref[...] = fl_v


def _topk_call(hm3):
    return pl.pallas_call(
        _topk_body,
        in_specs=[pl.BlockSpec((_B, _H, _W), lambda: (0, 0, 0))],
        out_specs=[
            pl.BlockSpec((_B, _KPAD), lambda: (0, 0)),
            pl.BlockSpec((_B, _KPAD), lambda: (0, 0)),
        ],
        out_shape=[
            jax.ShapeDtypeStruct((_B, _KPAD), jnp.float32),
            jax.ShapeDtypeStruct((_B, _KPAD), jnp.int32),
        ],
        scratch_shapes=[
            pltpu.VMEM((_H, _W), jnp.float32) for _ in range(_B)],
    )(hm3)


_HALF = 64


def _decode_body(sc_hbm, fl_hbm, off_hbm, wh_hbm, ids_out, sco_out, bb_out,
                 sidx, sscore, pobuf, obuf, wbuf, idsb, scb,
                 x1b, y1b, x2b, y2b, sem1, sem2):
    wid = lax.axis_index("s") * 2 + lax.axis_index("c")
    b = wid // 2
    start = (wid % 2) * _HALF
    pltpu.sync_copy(fl_hbm.at[pl.ds(b * _KPAD + start, _HALF)], sidx)
    pltpu.sync_copy(sc_hbm.at[pl.ds(b * _KPAD + start, _HALF)], sscore)
    base = b * (2 * _H * _W)
    for k in range(_HALF // 16):
        idx = sidx[pl.ds(k * 16, 16)]
        pobuf[pl.ds(k * 16, 16)] = base + idx
        pobuf[pl.ds(_HALF + k * 16, 16)] = base + _H * _W + idx
    pltpu.async_copy(off_hbm.at[pobuf], obuf, sem1).wait()
    pltpu.async_copy(wh_hbm.at[pobuf], wbuf, sem2).wait()
    neg1 = jnp.float32(-1.0)
    for k in range(_HALF // 16):
        idx = sidx[pl.ds(k * 16, 16)]
        s = sscore[pl.ds(k * 16, 16)]
        y = lax.shift_right_logical(idx, 8)
        x = jnp.bitwise_and(idx, _W - 1)
        ox = obuf[pl.ds(k * 16, 16)]
        oy = obuf[pl.ds(_HALF + k * 16, 16)]
        ww = wbuf[pl.ds(k * 16, 16)]
        hh = wbuf[pl.ds(_HALF + k * 16, 16)]
        keep = s > _THRESH
        cx = x.astype(jnp.float32) + ox
        cy = y.astype(jnp.float32) + oy
        hw = ww * 0.5
        hh2 = hh * 0.5
        x1 = jnp.where(keep, cx - hw, neg1) * _SCALE
        y1 = jnp.where(keep, cy - hh2, neg1) * _SCALE
        x2 = jnp.where(keep, cx + hw, neg1) * _SCALE
        y2 = jnp.where(keep, cy + hh2, neg1) * _SCALE
        idsb[pl.ds(k * 16, 16)] = jnp.where(keep, jnp.float32(0.0), neg1)
        scb[pl.ds(k * 16, 16)] = jnp.where(keep, s, neg1)
        x1b[pl.ds(k * 16, 16)] = x1
        y1b[pl.ds(k * 16, 16)] = y1
        x2b[pl.ds(k * 16, 16)] = x2
        y2b[pl.ds(k * 16, 16)] = y2
    pos = b * _KPAD + start
    pltpu.sync_copy(idsb, ids_out.at[pl.ds(pos, _HALF)])
    pltpu.sync_copy(scb, sco_out.at[pl.ds(pos, _HALF)])
    n = _B * _KPAD
    pltpu.sync_copy(x1b, bb_out.at[pl.ds(pos, _HALF)])
    pltpu.sync_copy(y1b, bb_out.at[pl.ds(n + pos, _HALF)])
    pltpu.sync_copy(x2b, bb_out.at[pl.ds(2 * n + pos, _HALF)])
    pltpu.sync_copy(y2b, bb_out.at[pl.ds(3 * n + pos, _HALF)])


def _decode_call(*args):
    return functools.partial(
        pl.kernel,
        mesh=plsc.VectorSubcoreMesh(core_axis_name="c", subcore_axis_name="s"),
        compiler_params=pltpu.CompilerParams(use_tc_tiling_on_sc=False),
        out_type=[
            jax.ShapeDtypeStruct((_B * _KPAD,), jnp.float32),
            jax.ShapeDtypeStruct((_B * _KPAD,), jnp.float32),
            jax.ShapeDtypeStruct((_B * _KPAD * 4,), jnp.float32),
        ],
        scratch_types=[
            pltpu.VMEM((_HALF,), jnp.int32),
            pltpu.VMEM((_HALF,), jnp.float32),
            pltpu.VMEM((2 * _HALF,), jnp.int32),
            pltpu.VMEM((2 * _HALF,), jnp.float32),
            pltpu.VMEM((2 * _HALF,), jnp.float32),
            pltpu.VMEM((_HALF,), jnp.float32),
            pltpu.VMEM((_HALF,), jnp.float32),
            pltpu.VMEM((_HALF,), jnp.float32),
            pltpu.VMEM((_HALF,), jnp.float32),
            pltpu.VMEM((_HALF,), jnp.float32),
            pltpu.VMEM((_HALF,), jnp.float32),
            pltpu.SemaphoreType.DMA,
            pltpu.SemaphoreType.DMA,
        ],
    )(_decode_body)(*args)


@jax.jit
def kernel(heatmap, offset, wh):
    hm3 = heatmap.reshape(_B, _H, _W)
    sc, fl = _topk_call(hm3)
    off_rows = offset.reshape(-1)
    wh_rows = wh.reshape(-1)
    ids_p, sco_p, bb_p = _decode_call(
        sc.reshape(-1), fl.reshape(-1), off_rows, wh_rows)
    ids = ids_p.reshape(_B, _KPAD)[:, :_TOPK, None]
    scores = sco_p.reshape(_B, _KPAD)[:, :_TOPK, None]
    bboxes = jnp.transpose(
        bb_p.reshape(4, _B, _KPAD), (1, 2, 0))[:, :_TOPK, :]
    return ids, scores, bboxes
